# baseline (speedup 1.0000x reference)
import jax
import jax.numpy as jnp
from jax.experimental import pallas as pl
from jax.experimental.pallas import tpu as pltpu

N_REF = 100000
K = 1000
CHUNK = 4096
NC = 25
NPAD = NC * CHUNK
B = 8
P = 6
FULL = 3
NQ = 1024
D = 128
DY = 64


def _encode(x_ref, w_ref, h_ref):
    h = jnp.dot(x_ref[...], w_ref[...], preferred_element_type=jnp.float32)
    h_ref[...] = h / jnp.sqrt(jnp.sum(h * h, axis=1, keepdims=True))


def _norm_cols(xt_ref, xn_ref):
    xt = xt_ref[...]
    nrm2 = jnp.sum(xt * xt, axis=0, keepdims=True)
    xn_ref[...] = xt / jnp.sqrt(jnp.maximum(nrm2, 1e-30))


def _sim_chunk(h, xn, col0):
    s = jnp.dot(h, xn, preferred_element_type=jnp.float32)
    col = col0 + jax.lax.broadcasted_iota(jnp.int32, (1, CHUNK), 1)
    return jnp.where(col < N_REF, s, -2.0)


def _apply_update(lo, hi, clo, chi, tb, cnt):
    lov = lo[...]
    hiv = hi[...]
    t = tb[...]
    c = cnt[...]
    ge = c >= float(K)
    inf = jnp.float32(jnp.inf)
    cand_lo = jnp.max(jnp.where(ge, t, -inf), axis=1, keepdims=True)
    cand_clo = jnp.min(jnp.where(ge, c, inf), axis=1, keepdims=True)
    cand_hi = jnp.min(jnp.where(ge, inf, t), axis=1, keepdims=True)
    cand_chi = jnp.max(jnp.where(ge, -inf, c), axis=1, keepdims=True)
    take_lo = cand_lo > lov
    take_hi = cand_hi < hiv
    lo[...] = jnp.where(take_lo, cand_lo, lov)
    clo[...] = jnp.where(take_lo, cand_clo, clo[...])
    hi[...] = jnp.where(take_hi, cand_hi, hiv)
    chi[...] = jnp.where(take_hi, cand_chi, chi[...])


def _offset_cluster(t_hat, w, a, base, lov, hiv):
    bidx = jax.lax.broadcasted_iota(jnp.int32, (1, B), 1)
    kexp = jnp.where(bidx >= B // 2, bidx - B // 2, B // 2 - 1 - bidx)
    mag = jnp.where(kexp == 0, 1.0,
                    jnp.where(kexp == 1, base,
                              jnp.where(kexp == 2, base * base,
                                        base * base * base)))
    sign = jnp.where(bidx >= B // 2, 1.0, -1.0)
    return jnp.clip(t_hat + (w * a) * (sign * mag), lov, hiv)


def _place_uniform4(lo, hi):
    lov = lo[...]
    hiv = hi[...]
    bidx = jax.lax.broadcasted_iota(jnp.int32, (1, B), 1)
    t = lov + (hiv - lov) * 0.2 * (bidx.astype(jnp.float32) + 1.0)
    return jnp.where(bidx < 4, t, hiv)


def _place_log(lo, hi, clo, chi):
    lov = lo[...]
    hiv = hi[...]
    w = hiv - lov
    clov = jnp.maximum(clo[...], 1.0)
    num = jnp.log(clov * (1.0 / float(K)))
    den = jnp.log(clov / jnp.maximum(chi[...], 0.5))
    frac = jnp.clip(num / jnp.maximum(den, 1e-6), 0.0, 1.0)
    return _offset_cluster(lov + w * frac, w, 0.002, 5.0, lov, hiv)


def _place_linear(lo, hi, clo, chi):
    lov = lo[...]
    hiv = hi[...]
    w = hiv - lov
    n = jnp.maximum(clo[...] - chi[...], 1.0)
    pfr = jnp.clip((clo[...] - float(K)) / n, 0.0, 1.0)
    a = 2.0 * jnp.sqrt(jnp.maximum(pfr * (1.0 - pfr), 0.01) / n)
    return _offset_cluster(lov + w * pfr, w, a, 3.0, lov, hiv)


def _count_kernel(h_ref, xn_ref, tb0_ref, out_ref,
                  lo, hi, clo, chi, tb, cnt):
    p = pl.program_id(0)
    c = pl.program_id(1)

    @pl.when(jnp.logical_and(p == 0, c == 0))
    def _init():
        lo[...] = jnp.full((NQ, 1), -1.01, jnp.float32)
        hi[...] = jnp.full((NQ, 1), 1.01, jnp.float32)
        clo[...] = jnp.full((NQ, 1), float(N_REF), jnp.float32)
        chi[...] = jnp.zeros((NQ, 1), jnp.float32)
        cnt[...] = jnp.zeros((NQ, B), jnp.float32)
        tb[...] = jnp.broadcast_to(tb0_ref[...], (NQ, B))

    @pl.when(jnp.logical_and(p > 0, c == 0))
    def _advance():
        _apply_update(lo, hi, clo, chi, tb, cnt)
        cnt[...] = jnp.zeros((NQ, B), jnp.float32)
        tb_log = _place_log(lo, hi, clo, chi)
        tb_lin = _place_linear(lo, hi, clo, chi)
        tb_u4 = _place_uniform4(lo, hi)
        tb[...] = jnp.where(p == 1, tb_log,
                            jnp.where(p == 2, tb_lin, tb_u4))

    s = _sim_chunk(h_ref[...], xn_ref[...], c * CHUNK)
    tbv = tb[...]

    @pl.when(p < FULL)
    def _count8():
        cols = []
        for b in range(B):
            cols.append(jnp.sum(jnp.where(s >= tbv[:, b:b + 1], 1.0, 0.0),
                                axis=1))
        cnt[...] = cnt[...] + jnp.stack(cols, axis=1)

    @pl.when(p >= FULL)
    def _count4():
        cols = []
        for b in range(4):
            cols.append(jnp.sum(jnp.where(s >= tbv[:, b:b + 1], 1.0, 0.0),
                                axis=1))
        cols += [jnp.zeros((NQ,), jnp.float32)] * (B - 4)
        cnt[...] = cnt[...] + jnp.stack(cols, axis=1)

    @pl.when(jnp.logical_and(p == P - 1, c == NC - 1))
    def _final():
        _apply_update(lo, hi, clo, chi, tb, cnt)
        out_ref[...] = jnp.concatenate(
            [lo[...], hi[...], clo[...], chi[...]], axis=1)


def _pred_kernel(h_ref, xn_ref, yt_ref, prm_ref, out_ref, num, den):
    c = pl.program_id(0)

    @pl.when(c == 0)
    def _init():
        num[...] = jnp.zeros((NQ, DY), jnp.float32)
        den[...] = jnp.zeros((NQ, 1), jnp.float32)

    s = _sim_chunk(h_ref[...], xn_ref[...], c * CHUNK)
    prm = prm_ref[...]
    lov = prm[:, 0:1]
    hiv = prm[:, 1:2]
    alpha = (float(K) - prm[:, 3:4]) / jnp.maximum(
        prm[:, 2:3] - prm[:, 3:4], 1.0)
    e = jnp.exp(s - 1.0)
    sel = jnp.where(s >= hiv, 1.0, jnp.where(s >= lov, alpha, 0.0))
    w = e * sel
    num[...] += jnp.dot(w, yt_ref[...], preferred_element_type=jnp.float32)
    den[...] += jnp.sum(w, axis=1, keepdims=True)

    @pl.when(c == NC - 1)
    def _done():
        out_ref[...] = num[...] / den[...]


def kernel(x, W_enc, ref_x, ref_y):
    h = pl.pallas_call(
        _encode,
        out_shape=jax.ShapeDtypeStruct((NQ, D), jnp.float32),
    )(x, W_enc)

    xt = jnp.pad(ref_x.T, ((0, 0), (0, NPAD - N_REF)))
    yt = jnp.pad(ref_y, ((0, NPAD - N_REF), (0, 0)))
    tb0 = jnp.array([[0.05, 0.12, 0.17, 0.21, 0.25, 0.3, 0.4, 0.6]],
                    jnp.float32)

    xn = pl.pallas_call(
        _norm_cols,
        grid=(NC,),
        in_specs=[pl.BlockSpec((D, CHUNK), lambda c: (0, c))],
        out_specs=pl.BlockSpec((D, CHUNK), lambda c: (0, c)),
        out_shape=jax.ShapeDtypeStruct((D, NPAD), jnp.float32),
    )(xt)

    prm = pl.pallas_call(
        _count_kernel,
        grid=(P, NC),
        in_specs=[
            pl.BlockSpec((NQ, D), lambda p, c: (0, 0)),
            pl.BlockSpec((D, CHUNK), lambda p, c: (0, c)),
            pl.BlockSpec((1, B), lambda p, c: (0, 0)),
        ],
        out_specs=pl.BlockSpec((NQ, 4), lambda p, c: (0, 0)),
        out_shape=jax.ShapeDtypeStruct((NQ, 4), jnp.float32),
        scratch_shapes=[
            pltpu.VMEM((NQ, 1), jnp.float32),
            pltpu.VMEM((NQ, 1), jnp.float32),
            pltpu.VMEM((NQ, 1), jnp.float32),
            pltpu.VMEM((NQ, 1), jnp.float32),
            pltpu.VMEM((NQ, B), jnp.float32),
            pltpu.VMEM((NQ, B), jnp.float32),
        ],
    )(h, xn, tb0)

    pred = pl.pallas_call(
        _pred_kernel,
        grid=(NC,),
        in_specs=[
            pl.BlockSpec((NQ, D), lambda c: (0, 0)),
            pl.BlockSpec((D, CHUNK), lambda c: (0, c)),
            pl.BlockSpec((CHUNK, DY), lambda c: (c, 0)),
            pl.BlockSpec((NQ, 4), lambda c: (0, 0)),
        ],
        out_specs=pl.BlockSpec((NQ, DY), lambda c: (0, 0)),
        out_shape=jax.ShapeDtypeStruct((NQ, DY), jnp.float32),
        scratch_shapes=[
            pltpu.VMEM((NQ, DY), jnp.float32),
            pltpu.VMEM((NQ, 1), jnp.float32),
        ],
    )(h, xn, yt, prm)
    return pred


# speedup vs baseline: 1.0342x; 1.0342x over previous
import jax
import jax.numpy as jnp
from jax.experimental import pallas as pl
from jax.experimental.pallas import tpu as pltpu

N_REF = 100000
K = 1000
CHUNK = 4096
NC = 25
NPAD = NC * CHUNK
B = 8
P = 5
NQ = 1024
D = 128
DY = 64


def _encode(x_ref, w_ref, h_ref):
    h = jnp.dot(x_ref[...], w_ref[...], preferred_element_type=jnp.float32)
    h_ref[...] = h / jnp.sqrt(jnp.sum(h * h, axis=1, keepdims=True))


def _norm_cols(xt_ref, xn_ref):
    xt = xt_ref[...]
    nrm2 = jnp.sum(xt * xt, axis=0, keepdims=True)
    xn_ref[...] = xt / jnp.sqrt(jnp.maximum(nrm2, 1e-30))


def _sim_chunk(h, xn, col0):
    s = jnp.dot(h, xn, preferred_element_type=jnp.float32)
    col = col0 + jax.lax.broadcasted_iota(jnp.int32, (1, CHUNK), 1)
    return jnp.where(col < N_REF, s, -2.0)


def _apply_update(lo, hi, clo, chi, tb, cnt):
    lov = lo[...]
    hiv = hi[...]
    t = tb[...]
    c = cnt[...]
    ge = c >= float(K)
    inf = jnp.float32(jnp.inf)
    cand_lo = jnp.max(jnp.where(ge, t, -inf), axis=1, keepdims=True)
    cand_clo = jnp.min(jnp.where(ge, c, inf), axis=1, keepdims=True)
    cand_hi = jnp.min(jnp.where(ge, inf, t), axis=1, keepdims=True)
    cand_chi = jnp.max(jnp.where(ge, -inf, c), axis=1, keepdims=True)
    take_lo = cand_lo > lov
    take_hi = cand_hi < hiv
    lo[...] = jnp.where(take_lo, cand_lo, lov)
    clo[...] = jnp.where(take_lo, cand_clo, clo[...])
    hi[...] = jnp.where(take_hi, cand_hi, hiv)
    chi[...] = jnp.where(take_hi, cand_chi, chi[...])


def _offset_cluster(t_hat, w, a, base, lov, hiv):
    bidx = jax.lax.broadcasted_iota(jnp.int32, (1, B), 1)
    kexp = jnp.where(bidx >= B // 2, bidx - B // 2, B // 2 - 1 - bidx)
    mag = jnp.where(kexp == 0, 1.0,
                    jnp.where(kexp == 1, base,
                              jnp.where(kexp == 2, base * base,
                                        base * base * base)))
    sign = jnp.where(bidx >= B // 2, 1.0, -1.0)
    return jnp.clip(t_hat + (w * a) * (sign * mag), lov, hiv)


def _place_uniform(lo, hi):
    lov = lo[...]
    step = (hi[...] - lov) * (1.0 / (B + 1))
    bidx = jax.lax.broadcasted_iota(jnp.int32, (1, B), 1).astype(jnp.float32)
    return lov + step * (bidx + 1.0)


def _place_log(lo, hi, clo, chi):
    lov = lo[...]
    hiv = hi[...]
    w = hiv - lov
    clov = jnp.maximum(clo[...], 1.0)
    num = jnp.log(clov * (1.0 / float(K)))
    den = jnp.log(clov / jnp.maximum(chi[...], 0.5))
    frac = jnp.clip(num / jnp.maximum(den, 1e-6), 0.0, 1.0)
    return _offset_cluster(lov + w * frac, w, 0.002, 5.0, lov, hiv)


def _place_linear(lo, hi, clo, chi):
    lov = lo[...]
    hiv = hi[...]
    w = hiv - lov
    n = jnp.maximum(clo[...] - chi[...], 1.0)
    pfr = jnp.clip((clo[...] - float(K)) / n, 0.0, 1.0)
    a = 2.0 * jnp.sqrt(jnp.maximum(pfr * (1.0 - pfr), 0.01) / n)
    return _offset_cluster(lov + w * pfr, w, a, 3.0, lov, hiv)


def _count_kernel(h_ref, xn_ref, tb0_ref, out_ref,
                  lo, hi, clo, chi, tb, cnt):
    p = pl.program_id(0)
    c = pl.program_id(1)

    @pl.when(jnp.logical_and(p == 0, c == 0))
    def _init():
        lo[...] = jnp.full((NQ, 1), -1.01, jnp.float32)
        hi[...] = jnp.full((NQ, 1), 1.01, jnp.float32)
        clo[...] = jnp.full((NQ, 1), float(N_REF), jnp.float32)
        chi[...] = jnp.zeros((NQ, 1), jnp.float32)
        cnt[...] = jnp.zeros((NQ, B), jnp.float32)
        tb[...] = jnp.broadcast_to(tb0_ref[...], (NQ, B))

    @pl.when(jnp.logical_and(p > 0, c == 0))
    def _advance():
        _apply_update(lo, hi, clo, chi, tb, cnt)
        cnt[...] = jnp.zeros((NQ, B), jnp.float32)
        tb_log = _place_log(lo, hi, clo, chi)
        tb_lin = _place_linear(lo, hi, clo, chi)
        tb_uni = _place_uniform(lo, hi)
        tb[...] = jnp.where(p == 1, tb_log,
                            jnp.where(p == 2, tb_lin, tb_uni))

    s = _sim_chunk(h_ref[...], xn_ref[...], c * CHUNK)
    tbv = tb[...]
    acc = cnt[...]
    cols = []
    for b in range(B):
        cols.append(jnp.sum(jnp.where(s >= tbv[:, b:b + 1], 1.0, 0.0),
                            axis=1))
    cnt[...] = acc + jnp.stack(cols, axis=1)

    @pl.when(jnp.logical_and(p == P - 1, c == NC - 1))
    def _final():
        _apply_update(lo, hi, clo, chi, tb, cnt)
        out_ref[...] = jnp.concatenate(
            [lo[...], hi[...], clo[...], chi[...]], axis=1)


def _pred_kernel(h_ref, xn_ref, yt_ref, prm_ref, out_ref, num, den):
    c = pl.program_id(0)

    @pl.when(c == 0)
    def _init():
        num[...] = jnp.zeros((NQ, DY), jnp.float32)
        den[...] = jnp.zeros((NQ, 1), jnp.float32)

    s = _sim_chunk(h_ref[...], xn_ref[...], c * CHUNK)
    prm = prm_ref[...]
    lov = prm[:, 0:1]
    hiv = prm[:, 1:2]
    alpha = (float(K) - prm[:, 3:4]) / jnp.maximum(
        prm[:, 2:3] - prm[:, 3:4], 1.0)
    e = jnp.exp(s - 1.0)
    sel = jnp.where(s >= hiv, 1.0, jnp.where(s >= lov, alpha, 0.0))
    w = e * sel
    num[...] += jnp.dot(w, yt_ref[...], preferred_element_type=jnp.float32)
    den[...] += jnp.sum(w, axis=1, keepdims=True)

    @pl.when(c == NC - 1)
    def _done():
        out_ref[...] = num[...] / den[...]


def kernel(x, W_enc, ref_x, ref_y):
    h = pl.pallas_call(
        _encode,
        out_shape=jax.ShapeDtypeStruct((NQ, D), jnp.float32),
    )(x, W_enc)

    xt = jnp.pad(ref_x.T, ((0, 0), (0, NPAD - N_REF)))
    yt = jnp.pad(ref_y, ((0, NPAD - N_REF), (0, 0)))
    tb0 = jnp.array([[0.05, 0.12, 0.17, 0.21, 0.25, 0.3, 0.4, 0.6]],
                    jnp.float32)

    xn = pl.pallas_call(
        _norm_cols,
        grid=(NC,),
        in_specs=[pl.BlockSpec((D, CHUNK), lambda c: (0, c))],
        out_specs=pl.BlockSpec((D, CHUNK), lambda c: (0, c)),
        out_shape=jax.ShapeDtypeStruct((D, NPAD), jnp.float32),
    )(xt)

    prm = pl.pallas_call(
        _count_kernel,
        grid=(P, NC),
        in_specs=[
            pl.BlockSpec((NQ, D), lambda p, c: (0, 0)),
            pl.BlockSpec((D, CHUNK), lambda p, c: (0, c)),
            pl.BlockSpec((1, B), lambda p, c: (0, 0)),
        ],
        out_specs=pl.BlockSpec((NQ, 4), lambda p, c: (0, 0)),
        out_shape=jax.ShapeDtypeStruct((NQ, 4), jnp.float32),
        scratch_shapes=[
            pltpu.VMEM((NQ, 1), jnp.float32),
            pltpu.VMEM((NQ, 1), jnp.float32),
            pltpu.VMEM((NQ, 1), jnp.float32),
            pltpu.VMEM((NQ, 1), jnp.float32),
            pltpu.VMEM((NQ, B), jnp.float32),
            pltpu.VMEM((NQ, B), jnp.float32),
        ],
    )(h, xn, tb0)

    pred = pl.pallas_call(
        _pred_kernel,
        grid=(NC,),
        in_specs=[
            pl.BlockSpec((NQ, D), lambda c: (0, 0)),
            pl.BlockSpec((D, CHUNK), lambda c: (0, c)),
            pl.BlockSpec((CHUNK, DY), lambda c: (c, 0)),
            pl.BlockSpec((NQ, 4), lambda c: (0, 0)),
        ],
        out_specs=pl.BlockSpec((NQ, DY), lambda c: (0, 0)),
        out_shape=jax.ShapeDtypeStruct((NQ, DY), jnp.float32),
        scratch_shapes=[
            pltpu.VMEM((NQ, DY), jnp.float32),
            pltpu.VMEM((NQ, 1), jnp.float32),
        ],
    )(h, xn, yt, prm)
    return pred
